# Initial kernel scaffold; baseline (speedup 1.0000x reference)
import jax
import jax.numpy as jnp
from jax.experimental import pallas as pl


def kernel(x, emb_weight):
    raise NotImplementedError("write your pallas kernel here")


# speedup vs baseline: 1.2893x; 1.2893x over previous
import functools

import jax
import jax.numpy as jnp
from jax import lax
from jax.experimental import pallas as pl
from jax.experimental.pallas import tpu as pltpu
from jax.experimental.pallas import tpu_sc as plsc

N_EMBD = 1024
NUM_TOKENS = 8192
NC = 2
NS = 16
NW = NC * NS
BPW = NUM_TOKENS // NW
CH = 32
NCHUNK = BPW // CH
LANES = 16
VPR = N_EMBD // LANES

_mesh = plsc.VectorSubcoreMesh(core_axis_name="c", subcore_axis_name="s")


@functools.partial(
    pl.kernel,
    mesh=_mesh,
    out_type=jax.ShapeDtypeStruct((NUM_TOKENS, N_EMBD), jnp.float32),
    scratch_types=[
        pltpu.VMEM((BPW,), jnp.int32),
        pltpu.VMEM((CH, N_EMBD), jnp.float32),
        pltpu.VMEM((CH, N_EMBD), jnp.float32),
        pltpu.SemaphoreType.DMA,
        pltpu.SemaphoreType.DMA,
        pltpu.SemaphoreType.DMA,
        pltpu.SemaphoreType.DMA,
    ],
)
def _emb_double(table_hbm, idx_hbm, out_hbm, idx_v, buf0, buf1, g0, g1, s0, s1):
    wid = lax.axis_index("s") * NC + lax.axis_index("c")
    base = wid * BPW
    pltpu.sync_copy(idx_hbm.at[pl.ds(base, BPW)], idx_v)

    bufs = (buf0, buf1)
    gsems = (g0, g1)
    ssems = (s0, s1)

    def gather_copy(c, slot):
        return pltpu.make_async_copy(
            table_hbm.at[idx_v.at[pl.ds(c * CH, CH)]], bufs[slot], gsems[slot])

    def scatter_copy(c, slot):
        return pltpu.make_async_copy(
            bufs[slot], out_hbm.at[pl.ds(base + c * CH, CH)], ssems[slot])

    def double_rows(buf):
        def body(r, _):
            for j in range(VPR):
                sl = pl.ds(j * LANES, LANES)
                v = buf[r, sl]
                buf[r, sl] = v + v
            return ()
        lax.fori_loop(0, CH, body, ())

    gather_copy(0, 0).start()
    for c in range(NCHUNK):
        slot = c % 2
        if c + 1 < NCHUNK:
            nslot = (c + 1) % 2
            if c >= 1:
                scatter_copy(c - 1, nslot).wait()
            gather_copy(c + 1, nslot).start()
        gather_copy(c, slot).wait()
        double_rows(bufs[slot])
        scatter_copy(c, slot).start()
    scatter_copy(NCHUNK - 2, 0).wait()
    scatter_copy(NCHUNK - 1, 1).wait()


def kernel(x, emb_weight):
    return _emb_double(emb_weight, x.astype(jnp.int32))


# speedup vs baseline: 1.3791x; 1.0697x over previous
import functools

import jax
import jax.numpy as jnp
from jax import lax
from jax.experimental import pallas as pl
from jax.experimental.pallas import tpu as pltpu
from jax.experimental.pallas import tpu_sc as plsc

N_EMBD = 1024
NUM_TOKENS = 8192
NC = 2
NS = 16
NW = NC * NS
BPW = NUM_TOKENS // NW
CH = 32
NCHUNK = BPW // CH
LANES = 16
VPR = N_EMBD // LANES

_mesh = plsc.VectorSubcoreMesh(core_axis_name="c", subcore_axis_name="s")


@functools.partial(
    pl.kernel,
    mesh=_mesh,
    out_type=jax.ShapeDtypeStruct((NUM_TOKENS, N_EMBD), jnp.float32),
    scratch_types=[
        pltpu.VMEM((BPW,), jnp.int32),
        pltpu.VMEM((CH, N_EMBD), jnp.float32),
        pltpu.VMEM((CH, N_EMBD), jnp.float32),
        pltpu.VMEM((CH, N_EMBD), jnp.float32),
        pltpu.SemaphoreType.DMA,
        pltpu.SemaphoreType.DMA,
        pltpu.SemaphoreType.DMA,
        pltpu.SemaphoreType.DMA,
        pltpu.SemaphoreType.DMA,
        pltpu.SemaphoreType.DMA,
    ],
)
def _emb_double(table_hbm, idx_hbm, out_hbm, idx_v,
                buf0, buf1, buf2, g0, g1, g2, s0, s1, s2):
    wid = lax.axis_index("s") * NC + lax.axis_index("c")
    base = wid * BPW
    pltpu.sync_copy(idx_hbm.at[pl.ds(base, BPW)], idx_v)

    bufs = (buf0, buf1, buf2)
    gsems = (g0, g1, g2)
    ssems = (s0, s1, s2)
    NBUF = 3

    def gather_copy(c):
        b = c % NBUF
        return pltpu.make_async_copy(
            table_hbm.at[idx_v.at[pl.ds(c * CH, CH)]], bufs[b], gsems[b])

    def scatter_copy(c):
        b = c % NBUF
        return pltpu.make_async_copy(
            bufs[b], out_hbm.at[pl.ds(base + c * CH, CH)], ssems[b])

    def double_rows(buf):
        def body(r, _):
            for j in range(VPR):
                sl = pl.ds(j * LANES, LANES)
                v = buf[r, sl]
                buf[r, sl] = v + v
            return ()
        lax.fori_loop(0, CH, body, ())

    gather_copy(0).start()
    scat_waited = -1
    for c in range(NCHUNK):
        g = c + 1
        if g < NCHUNK:
            w = g - NBUF
            if w >= 0:
                scatter_copy(w).wait()
                scat_waited = w
            gather_copy(g).start()
        gather_copy(c).wait()
        double_rows(bufs[c % NBUF])
        scatter_copy(c).start()
    for w in range(scat_waited + 1, NCHUNK):
        scatter_copy(w).wait()


def kernel(x, emb_weight):
    return _emb_double(emb_weight, x.astype(jnp.int32))


# speedup vs baseline: 1.3901x; 1.0080x over previous
import functools

import jax
import jax.numpy as jnp
from jax import lax
from jax.experimental import pallas as pl
from jax.experimental.pallas import tpu as pltpu
from jax.experimental.pallas import tpu_sc as plsc

N_EMBD = 1024
NUM_TOKENS = 8192
NC = 2
NS = 16
NW = NC * NS
BPW = NUM_TOKENS // NW
CH = 16
NCHUNK = BPW // CH
NBUF = 6
PRIME = 3
LANES = 16
VPR = N_EMBD // LANES

_mesh = plsc.VectorSubcoreMesh(core_axis_name="c", subcore_axis_name="s")


@functools.partial(
    pl.kernel,
    mesh=_mesh,
    out_type=jax.ShapeDtypeStruct((NUM_TOKENS, N_EMBD), jnp.float32),
    scratch_types=(
        [pltpu.VMEM((BPW,), jnp.int32)]
        + [pltpu.VMEM((CH, N_EMBD), jnp.float32)] * NBUF
        + [pltpu.SemaphoreType.DMA] * (2 * NBUF)
    ),
)
def _emb_double(table_hbm, idx_hbm, out_hbm, idx_v, *bufs_sems):
    bufs = bufs_sems[:NBUF]
    gsems = bufs_sems[NBUF:2 * NBUF]
    ssems = bufs_sems[2 * NBUF:]

    wid = lax.axis_index("s") * NC + lax.axis_index("c")
    base = wid * BPW
    pltpu.sync_copy(idx_hbm.at[pl.ds(base, BPW)], idx_v)

    def gather_copy(c):
        b = c % NBUF
        return pltpu.make_async_copy(
            table_hbm.at[idx_v.at[pl.ds(c * CH, CH)]], bufs[b], gsems[b])

    def scatter_copy(c):
        b = c % NBUF
        return pltpu.make_async_copy(
            bufs[b], out_hbm.at[pl.ds(base + c * CH, CH)], ssems[b])

    def double_rows(buf):
        def body(r, _):
            for j in range(VPR):
                sl = pl.ds(j * LANES, LANES)
                v = buf[r, sl]
                buf[r, sl] = v + v
            return ()
        lax.fori_loop(0, CH, body, ())

    for c in range(PRIME):
        gather_copy(c).start()
    scat_waited = -1
    for c in range(NCHUNK):
        g = c + PRIME
        if g < NCHUNK:
            w = g - NBUF
            if w >= 0:
                scatter_copy(w).wait()
                scat_waited = w
            gather_copy(g).start()
        gather_copy(c).wait()
        double_rows(bufs[c % NBUF])
        scatter_copy(c).start()
    for w in range(scat_waited + 1, NCHUNK):
        scatter_copy(w).wait()


def kernel(x, emb_weight):
    return _emb_double(emb_weight, x.astype(jnp.int32))
